# baseline (speedup 1.0000x reference)
import functools

import jax
import jax.numpy as jnp
from jax import lax
from jax.experimental import pallas as pl
from jax.experimental.pallas import tpu as pltpu
from jax.experimental.pallas import tpu_sc as plsc

D = 1024
NC = 2
NS = 16
NW = NC * NS
C = 16
NBUF = 4
LA = 2


def _sc_gather(idx_flat, pe):
    b_total = idx_flat.shape[0]
    bpw = b_total // NW
    nchunk = bpw // C
    niter = nchunk // NBUF

    mesh = plsc.VectorSubcoreMesh(core_axis_name="c", subcore_axis_name="s")

    @functools.partial(
        pl.kernel,
        mesh=mesh,
        out_type=jax.ShapeDtypeStruct((b_total, D), jnp.float32),
        scratch_types=[
            pltpu.VMEM((bpw,), jnp.int32),
        ]
        + [pltpu.VMEM((C, D), jnp.float32) for _ in range(NBUF)]
        + [pltpu.SemaphoreType.DMA for _ in range(2 * NBUF)],
    )
    def k(idx_hbm, table_hbm, out_hbm, idx_v, *rest):
        bufs = rest[:NBUF]
        gsems = rest[NBUF:2 * NBUF]
        psems = rest[2 * NBUF:]
        wid = lax.axis_index("s") * NC + lax.axis_index("c")
        base = wid * bpw
        pltpu.sync_copy(idx_hbm.at[pl.ds(base, bpw)], idx_v)

        def gather(w, j):
            pltpu.async_copy(
                table_hbm.at[idx_v.at[pl.ds(w * C, C)]], bufs[j], gsems[j]
            )

        def put(v, j):
            pltpu.async_copy(bufs[j], out_hbm.at[pl.ds(base + v * C, C)],
                             psems[j])

        def wait_g(j):
            pltpu.make_async_copy(table_hbm.at[pl.ds(0, C)], bufs[j],
                                  gsems[j]).wait()

        def wait_p(j):
            pltpu.make_async_copy(bufs[j], out_hbm.at[pl.ds(0, C)],
                                  psems[j]).wait()

        for w in range(LA):
            gather(w, w)

        for b in range(NBUF):
            wait_g(b)
            put(b, b)
            w = b + LA
            if w >= NBUF:
                wait_p(w % NBUF)
            gather(w, w % NBUF)

        def body(i, carry):
            for b in range(NBUF):
                v = NBUF * i + b
                wait_g(b)
                put(v, b)
                bw = (b + LA) % NBUF
                wait_p(bw)
                gather(v + LA, bw)
            return carry

        lax.fori_loop(1, niter - 1, body, 0)

        for b in range(NBUF):
            v = nchunk - NBUF + b
            wait_g(b)
            put(v, b)
            if b + LA < NBUF:
                bw = (b + LA) % NBUF
                wait_p(bw)
                gather(v + LA, bw)

        for b in range(NBUF):
            wait_p(b)

    return k(idx_flat, pe)


R_TC = 128
NV = 8192


def _tc_gather(idx_flat, pe):
    n = idx_flat.shape[0]

    def body(idx_ref, pe_ref, out_ref):
        i = pl.program_id(0)
        for r in range(R_TC):
            v = idx_ref[i * R_TC + r]
            out_ref[pl.ds(r, 1), :] = pe_ref[pl.ds(v, 1), :]

    grid_spec = pltpu.PrefetchScalarGridSpec(
        num_scalar_prefetch=1,
        grid=(n // R_TC,),
        in_specs=[pl.BlockSpec((NV, D), lambda i, idx_ref: (0, 0))],
        out_specs=pl.BlockSpec((R_TC, D), lambda i, idx_ref: (i, 0)),
    )
    return pl.pallas_call(
        body,
        grid_spec=grid_spec,
        out_shape=jax.ShapeDtypeStruct((n, D), jnp.float32),
    )(idx_flat, pe)


T_TC = 12288


def kernel(idxes, pe):
    idx_flat = idxes.reshape(-1).astype(jnp.int32)
    out_tc = _tc_gather(idx_flat[:T_TC], pe)
    out_sc = _sc_gather(idx_flat[T_TC:], pe)
    out = jnp.concatenate([out_tc, out_sc], axis=0)
    return out.reshape(idxes.shape + (D,))


# speedup vs baseline: 1.8433x; 1.8433x over previous
import functools

import jax
import jax.numpy as jnp
from jax import lax
from jax.experimental import pallas as pl
from jax.experimental.pallas import tpu as pltpu
from jax.experimental.pallas import tpu_sc as plsc

D = 1024
NC = 2
NS = 16
NW = NC * NS
C = 16
NBUF = 4
LA = 2


def _sc_gather(idx_flat, pe):
    b_total = idx_flat.shape[0]
    bpw = b_total // NW
    nchunk = bpw // C
    niter = nchunk // NBUF

    mesh = plsc.VectorSubcoreMesh(core_axis_name="c", subcore_axis_name="s")

    @functools.partial(
        pl.kernel,
        mesh=mesh,
        out_type=jax.ShapeDtypeStruct((b_total, D), jnp.float32),
        scratch_types=[
            pltpu.VMEM((bpw,), jnp.int32),
        ]
        + [pltpu.VMEM((C, D), jnp.float32) for _ in range(NBUF)]
        + [pltpu.SemaphoreType.DMA for _ in range(2 * NBUF)],
    )
    def k(idx_hbm, table_hbm, out_hbm, idx_v, *rest):
        bufs = rest[:NBUF]
        gsems = rest[NBUF:2 * NBUF]
        psems = rest[2 * NBUF:]
        wid = lax.axis_index("s") * NC + lax.axis_index("c")
        base = wid * bpw
        pltpu.sync_copy(idx_hbm.at[pl.ds(base, bpw)], idx_v)

        def gather(w, j):
            pltpu.async_copy(
                table_hbm.at[idx_v.at[pl.ds(w * C, C)]], bufs[j], gsems[j]
            )

        def put(v, j):
            pltpu.async_copy(bufs[j], out_hbm.at[pl.ds(base + v * C, C)],
                             psems[j])

        def wait_g(j):
            pltpu.make_async_copy(table_hbm.at[pl.ds(0, C)], bufs[j],
                                  gsems[j]).wait()

        def wait_p(j):
            pltpu.make_async_copy(bufs[j], out_hbm.at[pl.ds(0, C)],
                                  psems[j]).wait()

        for w in range(LA):
            gather(w, w)

        for b in range(NBUF):
            wait_g(b)
            put(b, b)
            w = b + LA
            if w >= NBUF:
                wait_p(w % NBUF)
            gather(w, w % NBUF)

        def body(i, carry):
            for b in range(NBUF):
                v = NBUF * i + b
                wait_g(b)
                put(v, b)
                bw = (b + LA) % NBUF
                wait_p(bw)
                gather(v + LA, bw)
            return carry

        lax.fori_loop(1, niter - 1, body, 0)

        for b in range(NBUF):
            v = nchunk - NBUF + b
            wait_g(b)
            put(v, b)
            if b + LA < NBUF:
                bw = (b + LA) % NBUF
                wait_p(bw)
                gather(v + LA, bw)

        for b in range(NBUF):
            wait_p(b)

    return k(idx_flat, pe)


def kernel(idxes, pe):
    idx_flat = idxes.reshape(-1).astype(jnp.int32)
    out = _sc_gather(idx_flat, pe)
    return out.reshape(idxes.shape + (D,))


# speedup vs baseline: 1.8471x; 1.0021x over previous
import functools

import jax
import jax.numpy as jnp
from jax import lax
from jax.experimental import pallas as pl
from jax.experimental.pallas import tpu as pltpu
from jax.experimental.pallas import tpu_sc as plsc

D = 1024
NC = 2
NS = 16
NW = NC * NS
C = 16
NBUF = 4
LA = 2


def _sc_gather(idxes, pe):
    nrow, ncol = idxes.shape
    b_total = nrow * ncol
    bpw = b_total // NW
    wpr = ncol // bpw
    nchunk = bpw // C
    niter = nchunk // NBUF

    mesh = plsc.VectorSubcoreMesh(core_axis_name="c", subcore_axis_name="s")

    @functools.partial(
        pl.kernel,
        mesh=mesh,
        out_type=jax.ShapeDtypeStruct((b_total, D), jnp.float32),
        scratch_types=[
            pltpu.VMEM((bpw,), jnp.int32),
        ]
        + [pltpu.VMEM((C, D), jnp.float32) for _ in range(NBUF)]
        + [pltpu.SemaphoreType.DMA for _ in range(2 * NBUF)],
    )
    def k(idx_hbm, table_hbm, out_hbm, idx_v, *rest):
        bufs = rest[:NBUF]
        gsems = rest[NBUF:2 * NBUF]
        psems = rest[2 * NBUF:]
        wid = lax.axis_index("s") * NC + lax.axis_index("c")
        base = wid * bpw
        pltpu.sync_copy(
            idx_hbm.at[wid // wpr, pl.ds((wid % wpr) * bpw, bpw)], idx_v
        )

        def gather(w, j):
            pltpu.async_copy(
                table_hbm.at[idx_v.at[pl.ds(w * C, C)]], bufs[j], gsems[j]
            )

        def put(v, j):
            pltpu.async_copy(bufs[j], out_hbm.at[pl.ds(base + v * C, C)],
                             psems[j])

        def wait_g(j):
            pltpu.make_async_copy(table_hbm.at[pl.ds(0, C)], bufs[j],
                                  gsems[j]).wait()

        def wait_p(j):
            pltpu.make_async_copy(bufs[j], out_hbm.at[pl.ds(0, C)],
                                  psems[j]).wait()

        for w in range(LA):
            gather(w, w)

        for b in range(NBUF):
            wait_g(b)
            put(b, b)
            w = b + LA
            if w >= NBUF:
                wait_p(w % NBUF)
            gather(w, w % NBUF)

        def body(i, carry):
            for b in range(NBUF):
                v = NBUF * i + b
                wait_g(b)
                put(v, b)
                bw = (b + LA) % NBUF
                wait_p(bw)
                gather(v + LA, bw)
            return carry

        lax.fori_loop(1, niter - 1, body, 0)

        for b in range(NBUF):
            v = nchunk - NBUF + b
            wait_g(b)
            put(v, b)
            if b + LA < NBUF:
                bw = (b + LA) % NBUF
                wait_p(bw)
                gather(v + LA, bw)

        for b in range(NBUF):
            wait_p(b)

    return k(idxes, pe)


def kernel(idxes, pe):
    out = _sc_gather(idxes.astype(jnp.int32), pe)
    return out.reshape(idxes.shape + (D,))
